# Initial kernel scaffold; baseline (speedup 1.0000x reference)
import jax
import jax.numpy as jnp
from jax.experimental import pallas as pl


def kernel(ground_atoms, clause_weight):
    raise NotImplementedError("write your pallas kernel here")


# speedup vs baseline: 1.5609x; 1.5609x over previous
import jax
import jax.numpy as jnp
import numpy as np
from jax.experimental import pallas as pl
from jax.experimental.pallas import tpu as pltpu

_NUM_PREDICATES = 256
_BATCH = 65536
_GATHER_IDX = np.array([0, 17, 42, 100, 128, 200, 255, 60], dtype=np.int32)
_SIGNS = np.array([-1.0, 1.0, -1.0, 1.0, -1.0, 1.0, -1.0, 1.0], dtype=np.float32)
_L = 8
_MIN_W = 0.0
_MAX_W = 500.0

_SVEC = np.zeros((_NUM_PREDICATES,), np.float32)
_SVEC[_GATHER_IDX] = _SIGNS
_ONEHOT = np.zeros((_NUM_PREDICATES, _L), np.float32)
_ONEHOT[_GATHER_IDX, np.arange(_L)] = 1.0

_BLOCK_B = 1024


def _body(w_ref, x_ref, svec_ref, onehot_ref, out_ref, delta_ref):
    x = x_ref[...]
    svec = svec_ref[...]
    sel = svec != 0.0
    z = x * svec
    zm = jnp.where(sel, z, -jnp.inf)
    m = jnp.max(zm, axis=-1, keepdims=True)
    e = jnp.where(sel, jnp.exp(z - m), 0.0)
    s = jnp.sum(e, axis=-1, keepdims=True)
    w = jnp.clip(w_ref[0], _MIN_W, _MAX_W)
    scat = (w * svec) * (e / s)
    out_ref[...] = scat
    delta_ref[...] = jax.lax.dot_general(
        scat, onehot_ref[...], (((1,), (0,)), ((), ())),
        preferred_element_type=jnp.float32)


def kernel(ground_atoms, clause_weight):
    w = jnp.reshape(clause_weight.astype(jnp.float32), (1,))
    grid = (_BATCH // _BLOCK_B,)
    out_shapes = (
        jax.ShapeDtypeStruct((_BATCH, _NUM_PREDICATES), jnp.float32),
        jax.ShapeDtypeStruct((_BATCH, _L), jnp.float32),
    )
    scattered, delta = pl.pallas_call(
        _body,
        grid=grid,
        in_specs=[
            pl.BlockSpec(memory_space=pltpu.SMEM),
            pl.BlockSpec((_BLOCK_B, _NUM_PREDICATES), lambda i: (i, 0)),
            pl.BlockSpec((1, _NUM_PREDICATES), lambda i: (0, 0)),
            pl.BlockSpec((_NUM_PREDICATES, _L), lambda i: (0, 0)),
        ],
        out_specs=(
            pl.BlockSpec((_BLOCK_B, _NUM_PREDICATES), lambda i: (i, 0)),
            pl.BlockSpec((_BLOCK_B, _L), lambda i: (i, 0)),
        ),
        out_shape=out_shapes,
    )(w, ground_atoms, jnp.asarray(_SVEC)[None, :], jnp.asarray(_ONEHOT))
    return scattered, delta
